# Initial kernel scaffold; baseline (speedup 1.0000x reference)
import jax
import jax.numpy as jnp
from jax.experimental import pallas as pl


def kernel(indices, table):
    raise NotImplementedError("write your pallas kernel here")


# speedup vs baseline: 4.6993x; 4.6993x over previous
import functools

import jax
import jax.numpy as jnp
from jax import lax
from jax.experimental import pallas as pl
from jax.experimental.pallas import tpu as pltpu
from jax.experimental.pallas import tpu_sc as plsc

EMBED_DIM = 32
GROUP = 128
K = 8
SLAB = K * GROUP
NC = 2
NS = 16
NW = NC * NS


@functools.partial(jax.jit, static_argnums=(2,))
def _sc_gather(idx_rows, table, n_slabs_per_w):
    B = idx_rows.shape[0] * GROUP
    mesh = plsc.VectorSubcoreMesh(core_axis_name="c", subcore_axis_name="s")

    @functools.partial(
        pl.kernel,
        out_type=jax.ShapeDtypeStruct((B, EMBED_DIM), jnp.float32),
        mesh=mesh,
        scratch_types=[
            pltpu.VMEM((K, GROUP), jnp.int32),
            pltpu.VMEM((SLAB, EMBED_DIM), jnp.float32),
            pltpu.SemaphoreType.DMA,
        ],
        compiler_params=pltpu.CompilerParams(use_tc_tiling_on_sc=False),
    )
    def k(idx_hbm, table_hbm, out_hbm, idx_v, rows_v, gsem):
        wid = lax.axis_index("s") * NC + lax.axis_index("c")
        row0 = wid * n_slabs_per_w * K

        @pl.loop(0, n_slabs_per_w)
        def slab_loop(i):
            r = row0 + i * K
            pltpu.sync_copy(idx_hbm.at[pl.ds(r, K)], idx_v)
            for j in range(K):
                pltpu.async_copy(
                    table_hbm.at[idx_v.at[j]],
                    rows_v.at[pl.ds(j * GROUP, GROUP)],
                    gsem)
            for j in range(K):
                pltpu.make_async_copy(
                    table_hbm.at[idx_v.at[j]],
                    rows_v.at[pl.ds(j * GROUP, GROUP)],
                    gsem).wait()
            pltpu.sync_copy(rows_v, out_hbm.at[pl.ds(r * GROUP, SLAB)])

    return k(idx_rows, table)


def kernel(indices, table):
    Bt, S = indices.shape
    B = Bt * S
    idx_rows = indices.reshape(B // GROUP, GROUP).astype(jnp.int32)
    n_slabs_per_w = B // (NW * SLAB)
    out = _sc_gather(idx_rows, table, n_slabs_per_w)
    return out.reshape(Bt, S, EMBED_DIM)


# speedup vs baseline: 4.8391x; 1.0297x over previous
import functools

import jax
import jax.numpy as jnp
from jax import lax
from jax.experimental import pallas as pl
from jax.experimental.pallas import tpu as pltpu
from jax.experimental.pallas import tpu_sc as plsc

EMBED_DIM = 32
GROUP = 128
K = 8
SLAB = K * GROUP
NC = 2
NS = 16
NW = NC * NS


@functools.partial(jax.jit, static_argnums=(2,))
def _sc_gather(idx_rows, table, n_slabs_per_w):
    B = idx_rows.shape[0] * GROUP
    mesh = plsc.VectorSubcoreMesh(core_axis_name="c", subcore_axis_name="s")

    @functools.partial(
        pl.kernel,
        out_type=jax.ShapeDtypeStruct((B, EMBED_DIM), jnp.float32),
        mesh=mesh,
        scratch_types=[
            pltpu.VMEM((2, K, GROUP), jnp.int32),
            pltpu.VMEM((2, SLAB, EMBED_DIM), jnp.float32),
            pltpu.SemaphoreType.DMA,
            pltpu.SemaphoreType.DMA,
            pltpu.SemaphoreType.DMA,
            pltpu.SemaphoreType.DMA,
        ],
        compiler_params=pltpu.CompilerParams(use_tc_tiling_on_sc=False),
    )
    def k(idx_hbm, table_hbm, out_hbm, idx_v, rows_v, g0, g1, o0, o1):
        gsems = (g0, g1)
        osems = (o0, o1)
        wid = lax.axis_index("s") * NC + lax.axis_index("c")
        row0 = wid * n_slabs_per_w * K

        def stage_and_fire(slab, buf):
            r = row0 + slab * K
            pltpu.sync_copy(idx_hbm.at[pl.ds(r, K)], idx_v.at[buf])
            for j in range(K):
                pltpu.async_copy(
                    table_hbm.at[idx_v.at[buf, j]],
                    rows_v.at[buf, pl.ds(j * GROUP, GROUP)],
                    gsems[buf])

        def wait_gathers(buf):
            for j in range(K):
                pltpu.make_async_copy(
                    table_hbm.at[idx_v.at[buf, j]],
                    rows_v.at[buf, pl.ds(j * GROUP, GROUP)],
                    gsems[buf]).wait()

        def fire_write(slab, buf):
            r = row0 + slab * K
            pltpu.async_copy(
                rows_v.at[buf], out_hbm.at[pl.ds(r * GROUP, SLAB)], osems[buf])

        def wait_write(slab, buf):
            r = row0 + slab * K
            pltpu.make_async_copy(
                rows_v.at[buf], out_hbm.at[pl.ds(r * GROUP, SLAB)],
                osems[buf]).wait()

        n_pairs = n_slabs_per_w // 2
        stage_and_fire(0, 0)

        @pl.loop(0, n_pairs)
        def pair_loop(i):
            s0 = 2 * i

            @pl.when(i > 0)
            def _():
                wait_write(s0 - 1, 1)
            stage_and_fire(s0 + 1, 1)
            wait_gathers(0)
            fire_write(s0, 0)

            @pl.when(i < n_pairs - 1)
            def _():
                wait_write(s0, 0)
                stage_and_fire(s0 + 2, 0)
            wait_gathers(1)
            fire_write(s0 + 1, 1)

        wait_write(n_slabs_per_w - 2, 0)
        wait_write(n_slabs_per_w - 1, 1)

    return k(idx_rows, table)


def kernel(indices, table):
    Bt, S = indices.shape
    B = Bt * S
    idx_rows = indices.reshape(B // GROUP, GROUP).astype(jnp.int32)
    n_slabs_per_w = B // (NW * SLAB)
    out = _sc_gather(idx_rows, table, n_slabs_per_w)
    return out.reshape(Bt, S, EMBED_DIM)
